# baseline (speedup 1.0000x reference)
import jax
import jax.numpy as jnp
from jax.experimental import pallas as pl
from jax.experimental.pallas import tpu as pltpu

_B, _L, _D = 8, 2048, 1024
_W = 128
_S = 512
_SB = _S // _W
_NS = _L // _S
_SF = _S + 32
_WF = _W + 16


def _hs_copy(hs_any, hs_vmem, sem, bb, c0, buf):
    c0a = jnp.minimum((c0 // 8) * 8, _L - _SF)
    return pltpu.make_async_copy(
        hs_any.at[bb, pl.ds(c0a, _SF), :], hs_vmem.at[buf], sem.at[buf])


def _body(counts_smem, mask_ref, prob_ref, res_ref, state_ref, hs_any,
          out_ref, newstate_ref, h_ref, c0_ref, did_ref, hs_vmem, sem):
    b = pl.program_id(0)
    i = pl.program_id(1)
    buf = jax.lax.rem(i, 2)

    @pl.when((b == 0) & (i == 0))
    def _first():
        first_need = counts_smem[0] > 0
        did_ref[0] = first_need.astype(jnp.int32)

        @pl.when(first_need)
        def _():
            _hs_copy(hs_any, hs_vmem, sem, 0, 0, 0).start()

    @pl.when(i == 0)
    def _init_h():
        c0_ref[0] = 0
        h_ref[...] = state_ref[pl.ds(b, 1), :]

    c0 = c0_ref[0]
    cnt = counts_smem[b]

    mask_panel = mask_ref[0, 0]
    p_panel = prob_ref[0, 0]
    nsel_panel = jnp.sum(mask_panel)
    c0n = c0 + nsel_panel

    last_i = i == _NS - 1
    nb = jnp.where(last_i, b + 1, b)
    nc0 = jnp.where(last_i, 0, c0n)
    ncnt = counts_smem[jnp.minimum(nb, _B - 1)]
    neednext = jnp.logical_not((b == _B - 1) & last_i) & (nc0 < ncnt)
    did_ref[1 - buf] = neednext.astype(jnp.int32)

    @pl.when(neednext)
    def _prefetch():
        _hs_copy(hs_any, hs_vmem, sem, nb, nc0, 1 - buf).start()

    @pl.when(did_ref[buf] > 0)
    def _wait():
        _hs_copy(hs_any, hs_vmem, sem, b, c0, buf).wait()

    c0a = jnp.minimum((c0 // 8) * 8, _L - _SF)

    iota_r = jax.lax.broadcasted_iota(jnp.int32, (_W, _W), 0)
    iota_c = jax.lax.broadcasted_iota(jnp.int32, (_W, _W), 1)
    eye = (iota_r == iota_c).astype(jnp.float32)
    utones = (iota_r <= iota_c).astype(jnp.float32)
    col_r = jax.lax.broadcasted_iota(
        jnp.int32, (_W, _WF), 1).astype(jnp.float32)

    def tocol(xrow):
        return jnp.sum(eye * xrow, axis=1, keepdims=True)

    cs = c0
    for s in range(_SB):
        lo = s * _W
        mask_row = mask_panel[:, lo:lo + _W]
        p_row = p_panel[:, lo:lo + _W]

        incl_row = jnp.dot(mask_row.astype(jnp.float32), utones,
                           preferred_element_type=jnp.float32)
        incl_i = incl_row.astype(jnp.int32)
        pos_row = cs + incl_i - 1
        valid_row = ((pos_row >= 0) & (pos_row < cnt)).astype(jnp.float32)
        valid_col = tocol(valid_row)

        heavy = cs < cnt
        cs_cap = cs

        @pl.when(heavy)
        def _heavy(cs=cs_cap, mask_row=mask_row, p_row=p_row,
                   pos_row=pos_row, incl_i=incl_i, valid_col=valid_col,
                   lo=lo):
            active = (mask_row > 0) & (pos_row < cnt)
            a_row = jnp.where(active, 1.0 - p_row, 1.0)
            a_row = jnp.maximum(a_row, 1e-30)
            loga = jnp.log(a_row)
            st_row = jnp.dot(loga, utones, preferred_element_type=jnp.float32)
            st_col = tocol(st_row)
            smat = jnp.minimum(st_col - st_row, 0.0)
            lmat = jnp.where(iota_c <= iota_r, jnp.exp(smat), 0.0)

            loc = cs - c0a
            loc8 = jnp.minimum((loc // 8) * 8, _SF - _WF)
            off2 = loc - loc8
            pa_col = tocol(jnp.where(active, p_row, 0.0))
            rank_col = tocol((incl_i - 1 + off2).astype(jnp.float32))
            emat = (rank_col == col_r).astype(jnp.float32)
            m1 = jnp.dot(lmat, pa_col * emat,
                         preferred_element_type=jnp.float32)

            hs_sub = hs_vmem[buf, pl.ds(loc8, _WF), :]
            h_row = h_ref[...]
            p_carry = jnp.exp(jnp.minimum(st_col, 0.0))

            h_new = (jnp.dot(m1[_W - 1:_W, :], hs_sub,
                             preferred_element_type=jnp.float32)
                     + p_carry[_W - 1:_W] * h_row)
            h_ref[...] = h_new

            hmat = jnp.dot(m1, hs_sub, preferred_element_type=jnp.float32)
            hmat = hmat + p_carry * h_row

            out_ref[0, lo:lo + _W, :] = (
                res_ref[0, lo:lo + _W, :] + hmat * valid_col)

        @pl.when(jnp.logical_not(heavy))
        def _cheap(valid_col=valid_col, lo=lo):
            out_ref[0, lo:lo + _W, :] = (
                res_ref[0, lo:lo + _W, :] + valid_col * h_ref[...])

        cs = cs + jnp.sum(mask_row)

    c0_ref[0] = c0n

    @pl.when(last_i)
    def _fin():
        newstate_ref[pl.ds(b, 1), :] = h_ref[...]


def kernel(hidden_states, residual, token_mask, prob, counts, state):
    mask4 = token_mask.astype(jnp.int32).reshape(_B, _NS, 1, _S)
    prob4 = prob.astype(jnp.float32).reshape(_B, _NS, 1, _S)

    out, new_state = pl.pallas_call(
        _body,
        grid=(_B, _NS),
        in_specs=[
            pl.BlockSpec(memory_space=pltpu.SMEM),
            pl.BlockSpec((1, 1, 1, _S), lambda b, i: (b, i, 0, 0)),
            pl.BlockSpec((1, 1, 1, _S), lambda b, i: (b, i, 0, 0)),
            pl.BlockSpec((1, _S, _D), lambda b, i: (b, i, 0)),
            pl.BlockSpec((_B, _D), lambda b, i: (0, 0)),
            pl.BlockSpec(memory_space=pltpu.MemorySpace.HBM),
        ],
        out_specs=[
            pl.BlockSpec((1, _S, _D), lambda b, i: (b, i, 0)),
            pl.BlockSpec((_B, _D), lambda b, i: (0, 0)),
        ],
        out_shape=[
            jax.ShapeDtypeStruct((_B, _L, _D), jnp.float32),
            jax.ShapeDtypeStruct((_B, _D), jnp.float32),
        ],
        scratch_shapes=[
            pltpu.VMEM((1, _D), jnp.float32),
            pltpu.SMEM((1,), jnp.int32),
            pltpu.SMEM((2,), jnp.int32),
            pltpu.VMEM((2, _SF, _D), jnp.float32),
            pltpu.SemaphoreType.DMA((2,)),
        ],
        compiler_params=pltpu.CompilerParams(
            dimension_semantics=("arbitrary", "arbitrary"),
        ),
    )(counts.astype(jnp.int32), mask4, prob4, residual,
      state.astype(jnp.float32), hidden_states)

    return out, new_state


# speedup vs baseline: 1.0451x; 1.0451x over previous
import jax
import jax.numpy as jnp
from jax.experimental import pallas as pl
from jax.experimental.pallas import tpu as pltpu

_B, _L, _D = 8, 2048, 1024
_W = 128
_S = 512
_SB = _S // _W
_NS = _L // _S
_SF = _S + 32
_WF = _W + 16


def _hs_copy(hs_any, hs_vmem, sem, bb, c0, buf):
    c0a = jnp.minimum((c0 // 8) * 8, _L - _SF)
    return pltpu.make_async_copy(
        hs_any.at[bb, pl.ds(c0a, _SF), :], hs_vmem.at[buf], sem.at[buf])


def _body(counts_smem, mask_ref, prob_ref, res_ref, state_ref, hs_any,
          out_ref, newstate_ref, h_ref, c0_ref, did_ref, hs_vmem, sem):
    b = pl.program_id(0)
    i = pl.program_id(1)
    buf = jax.lax.rem(i, 2)

    @pl.when((b == 0) & (i == 0))
    def _first():
        first_need = counts_smem[0] > 0
        did_ref[0] = first_need.astype(jnp.int32)

        @pl.when(first_need)
        def _():
            _hs_copy(hs_any, hs_vmem, sem, 0, 0, 0).start()

    @pl.when(i == 0)
    def _init_h():
        c0_ref[0] = 0
        h_ref[...] = state_ref[pl.ds(b, 1), :]

    c0 = c0_ref[0]
    cnt = counts_smem[b]

    mask_panel = mask_ref[0, 0]
    p_panel = prob_ref[0, 0]
    nsel_panel = jnp.sum(mask_panel)
    c0n = c0 + nsel_panel

    last_i = i == _NS - 1
    nb = jnp.where(last_i, b + 1, b)
    nc0 = jnp.where(last_i, 0, c0n)
    ncnt = counts_smem[jnp.minimum(nb, _B - 1)]
    neednext = jnp.logical_not((b == _B - 1) & last_i) & (nc0 < ncnt)
    did_ref[1 - buf] = neednext.astype(jnp.int32)

    @pl.when(neednext)
    def _prefetch():
        _hs_copy(hs_any, hs_vmem, sem, nb, nc0, 1 - buf).start()

    @pl.when(did_ref[buf] > 0)
    def _wait():
        _hs_copy(hs_any, hs_vmem, sem, b, c0, buf).wait()

    c0a = jnp.minimum((c0 // 8) * 8, _L - _SF)

    iota_r = jax.lax.broadcasted_iota(jnp.int32, (_W, _W), 0)
    iota_c = jax.lax.broadcasted_iota(jnp.int32, (_W, _W), 1)
    eye = (iota_r == iota_c).astype(jnp.float32)
    utones = (iota_r <= iota_c).astype(jnp.float32)
    col_r = jax.lax.broadcasted_iota(
        jnp.int32, (_W, _WF), 1).astype(jnp.float32)

    def tocol(xrow):
        return jnp.sum(eye * xrow, axis=1, keepdims=True)

    cs = c0
    for s in range(_SB):
        lo = s * _W
        mask_row = mask_panel[:, lo:lo + _W]
        p_row = p_panel[:, lo:lo + _W]

        incl_row = jnp.dot(mask_row.astype(jnp.float32), utones,
                           preferred_element_type=jnp.float32)
        incl_i = incl_row.astype(jnp.int32)
        pos_row = cs + incl_i - 1
        valid_row = ((pos_row >= 0) & (pos_row < cnt)).astype(jnp.float32)
        valid_col = tocol(valid_row)

        heavy = cs < cnt
        cs_cap = cs

        @pl.when(heavy)
        def _heavy(cs=cs_cap, mask_row=mask_row, p_row=p_row,
                   pos_row=pos_row, incl_i=incl_i, valid_col=valid_col,
                   lo=lo):
            active = (mask_row > 0) & (pos_row < cnt)
            a_row = jnp.where(active, 1.0 - p_row, 1.0)
            a_row = jnp.maximum(a_row, 1e-30)
            loga = jnp.log(a_row)
            st_row = jnp.dot(loga, utones, preferred_element_type=jnp.float32)
            st_col = tocol(st_row)
            smat = jnp.minimum(st_col - st_row, 0.0)
            lmat = jnp.where(iota_c <= iota_r, jnp.exp(smat), 0.0)

            loc = cs - c0a
            loc8 = jnp.minimum((loc // 8) * 8, _SF - _WF)
            off2 = loc - loc8
            pa_col = tocol(jnp.where(active, p_row, 0.0))
            rank_col = tocol((incl_i - 1 + off2).astype(jnp.float32))
            emat = (rank_col == col_r).astype(jnp.float32)
            m1 = jnp.dot(lmat, pa_col * emat,
                         preferred_element_type=jnp.float32)

            hs_sub = hs_vmem[buf, pl.ds(loc8, _WF), :]
            h_row = h_ref[...]
            p_carry = jnp.exp(jnp.minimum(st_col, 0.0))
            hmat = jnp.dot(m1, hs_sub, preferred_element_type=jnp.float32)
            hmat = hmat + p_carry * h_row

            out_ref[0, lo:lo + _W, :] = (
                res_ref[0, lo:lo + _W, :] + hmat * valid_col)
            h_ref[...] = hmat[_W - 1:_W, :]

        @pl.when(jnp.logical_not(heavy))
        def _cheap(valid_col=valid_col, lo=lo):
            out_ref[0, lo:lo + _W, :] = (
                res_ref[0, lo:lo + _W, :] + valid_col * h_ref[...])

        cs = cs + jnp.sum(mask_row)

    c0_ref[0] = c0n

    @pl.when(last_i)
    def _fin():
        newstate_ref[pl.ds(b, 1), :] = h_ref[...]


def kernel(hidden_states, residual, token_mask, prob, counts, state):
    mask4 = token_mask.astype(jnp.int32).reshape(_B, _NS, 1, _S)
    prob4 = prob.astype(jnp.float32).reshape(_B, _NS, 1, _S)

    out, new_state = pl.pallas_call(
        _body,
        grid=(_B, _NS),
        in_specs=[
            pl.BlockSpec(memory_space=pltpu.SMEM),
            pl.BlockSpec((1, 1, 1, _S), lambda b, i: (b, i, 0, 0)),
            pl.BlockSpec((1, 1, 1, _S), lambda b, i: (b, i, 0, 0)),
            pl.BlockSpec((1, _S, _D), lambda b, i: (b, i, 0)),
            pl.BlockSpec((_B, _D), lambda b, i: (0, 0)),
            pl.BlockSpec(memory_space=pltpu.MemorySpace.HBM),
        ],
        out_specs=[
            pl.BlockSpec((1, _S, _D), lambda b, i: (b, i, 0)),
            pl.BlockSpec((_B, _D), lambda b, i: (0, 0)),
        ],
        out_shape=[
            jax.ShapeDtypeStruct((_B, _L, _D), jnp.float32),
            jax.ShapeDtypeStruct((_B, _D), jnp.float32),
        ],
        scratch_shapes=[
            pltpu.VMEM((1, _D), jnp.float32),
            pltpu.SMEM((1,), jnp.int32),
            pltpu.SMEM((2,), jnp.int32),
            pltpu.VMEM((2, _SF, _D), jnp.float32),
            pltpu.SemaphoreType.DMA((2,)),
        ],
        compiler_params=pltpu.CompilerParams(
            dimension_semantics=("arbitrary", "arbitrary"),
        ),
    )(counts.astype(jnp.int32), mask4, prob4, residual,
      state.astype(jnp.float32), hidden_states)

    return out, new_state


# speedup vs baseline: 1.0628x; 1.0169x over previous
import jax
import jax.numpy as jnp
from jax.experimental import pallas as pl
from jax.experimental.pallas import tpu as pltpu

_B, _L, _D = 8, 2048, 1024
_W = 128
_S = 1024
_SB = _S // _W
_NS = _L // _S
_SF = _S + 16
_WF = _W + 16


def _hs_copy(hs_any, hs_vmem, sem, bb, c0, buf):
    c0a = jnp.minimum((c0 // 8) * 8, _L - _SF)
    return pltpu.make_async_copy(
        hs_any.at[bb, pl.ds(c0a, _SF), :], hs_vmem.at[buf], sem.at[buf])


def _body(counts_smem, mask_ref, prob_ref, res_ref, state_ref, hs_any,
          out_ref, newstate_ref, h_ref, c0_ref, did_ref, hs_vmem, sem):
    b = pl.program_id(0)
    i = pl.program_id(1)
    buf = jax.lax.rem(i, 2)

    @pl.when((b == 0) & (i == 0))
    def _first():
        first_need = counts_smem[0] > 0
        did_ref[0] = first_need.astype(jnp.int32)

        @pl.when(first_need)
        def _():
            _hs_copy(hs_any, hs_vmem, sem, 0, 0, 0).start()

    @pl.when(i == 0)
    def _init_h():
        c0_ref[0] = 0
        h_ref[...] = state_ref[pl.ds(b, 1), :]

    c0 = c0_ref[0]
    cnt = counts_smem[b]

    mask_panel = mask_ref[0, 0]
    p_panel = prob_ref[0, 0]
    nsel_panel = jnp.sum(mask_panel)
    c0n = c0 + nsel_panel

    last_i = i == _NS - 1
    nb = jnp.where(last_i, b + 1, b)
    nc0 = jnp.where(last_i, 0, c0n)
    ncnt = counts_smem[jnp.minimum(nb, _B - 1)]
    neednext = jnp.logical_not((b == _B - 1) & last_i) & (nc0 < ncnt)
    did_ref[1 - buf] = neednext.astype(jnp.int32)

    @pl.when(neednext)
    def _prefetch():
        _hs_copy(hs_any, hs_vmem, sem, nb, nc0, 1 - buf).start()

    @pl.when(did_ref[buf] > 0)
    def _wait():
        _hs_copy(hs_any, hs_vmem, sem, b, c0, buf).wait()

    c0a = jnp.minimum((c0 // 8) * 8, _L - _SF)

    iota_r = jax.lax.broadcasted_iota(jnp.int32, (_W, _W), 0)
    iota_c = jax.lax.broadcasted_iota(jnp.int32, (_W, _W), 1)
    eye = (iota_r == iota_c).astype(jnp.float32)
    utones = (iota_r <= iota_c).astype(jnp.float32)
    col_r = jax.lax.broadcasted_iota(
        jnp.int32, (_W, _WF), 1).astype(jnp.float32)

    def tocol(xrow):
        return jnp.sum(eye * xrow, axis=1, keepdims=True)

    cs = c0
    for s in range(_SB):
        lo = s * _W
        mask_row = mask_panel[:, lo:lo + _W]
        p_row = p_panel[:, lo:lo + _W]

        incl_row = jnp.dot(mask_row.astype(jnp.float32), utones,
                           preferred_element_type=jnp.float32)
        incl_i = incl_row.astype(jnp.int32)
        pos_row = cs + incl_i - 1
        valid_row = ((pos_row >= 0) & (pos_row < cnt)).astype(jnp.float32)
        valid_col = tocol(valid_row)

        heavy = cs < cnt
        cs_cap = cs

        @pl.when(heavy)
        def _heavy(cs=cs_cap, mask_row=mask_row, p_row=p_row,
                   pos_row=pos_row, incl_i=incl_i, valid_col=valid_col,
                   lo=lo):
            active = (mask_row > 0) & (pos_row < cnt)
            a_row = jnp.where(active, 1.0 - p_row, 1.0)
            a_row = jnp.maximum(a_row, 1e-30)
            loga = jnp.log(a_row)
            st_row = jnp.dot(loga, utones, preferred_element_type=jnp.float32)
            st_col = tocol(st_row)
            smat = jnp.minimum(st_col - st_row, 0.0)
            lmat = jnp.where(iota_c <= iota_r, jnp.exp(smat), 0.0)

            loc = cs - c0a
            loc8 = jnp.minimum((loc // 8) * 8, _SF - _WF)
            off2 = loc - loc8
            pa_col = tocol(jnp.where(active, p_row, 0.0))
            rank_col = tocol((incl_i - 1 + off2).astype(jnp.float32))
            emat = (rank_col == col_r).astype(jnp.float32)
            m1 = jnp.dot(lmat, pa_col * emat,
                         preferred_element_type=jnp.float32)

            hs_sub = hs_vmem[buf, pl.ds(loc8, _WF), :]
            h_row = h_ref[...]
            p_carry = jnp.exp(jnp.minimum(st_col, 0.0))
            hmat = jnp.dot(m1, hs_sub, preferred_element_type=jnp.float32)
            hmat = hmat + p_carry * h_row

            out_ref[0, lo:lo + _W, :] = (
                res_ref[0, lo:lo + _W, :] + hmat * valid_col)
            h_ref[...] = hmat[_W - 1:_W, :]

        @pl.when(jnp.logical_not(heavy))
        def _cheap(valid_col=valid_col, lo=lo):
            out_ref[0, lo:lo + _W, :] = (
                res_ref[0, lo:lo + _W, :] + valid_col * h_ref[...])

        cs = cs + jnp.sum(mask_row)

    c0_ref[0] = c0n

    @pl.when(last_i)
    def _fin():
        newstate_ref[pl.ds(b, 1), :] = h_ref[...]


def kernel(hidden_states, residual, token_mask, prob, counts, state):
    mask4 = token_mask.astype(jnp.int32).reshape(_B, _NS, 1, _S)
    prob4 = prob.astype(jnp.float32).reshape(_B, _NS, 1, _S)

    out, new_state = pl.pallas_call(
        _body,
        grid=(_B, _NS),
        in_specs=[
            pl.BlockSpec(memory_space=pltpu.SMEM),
            pl.BlockSpec((1, 1, 1, _S), lambda b, i: (b, i, 0, 0)),
            pl.BlockSpec((1, 1, 1, _S), lambda b, i: (b, i, 0, 0)),
            pl.BlockSpec((1, _S, _D), lambda b, i: (b, i, 0)),
            pl.BlockSpec((_B, _D), lambda b, i: (0, 0)),
            pl.BlockSpec(memory_space=pltpu.MemorySpace.HBM),
        ],
        out_specs=[
            pl.BlockSpec((1, _S, _D), lambda b, i: (b, i, 0)),
            pl.BlockSpec((_B, _D), lambda b, i: (0, 0)),
        ],
        out_shape=[
            jax.ShapeDtypeStruct((_B, _L, _D), jnp.float32),
            jax.ShapeDtypeStruct((_B, _D), jnp.float32),
        ],
        scratch_shapes=[
            pltpu.VMEM((1, _D), jnp.float32),
            pltpu.SMEM((1,), jnp.int32),
            pltpu.SMEM((2,), jnp.int32),
            pltpu.VMEM((2, _SF, _D), jnp.float32),
            pltpu.SemaphoreType.DMA((2,)),
        ],
        compiler_params=pltpu.CompilerParams(
            dimension_semantics=("arbitrary", "arbitrary"),
        ),
    )(counts.astype(jnp.int32), mask4, prob4, residual,
      state.astype(jnp.float32), hidden_states)

    return out, new_state
